# Initial kernel scaffold; baseline (speedup 1.0000x reference)
import jax
import jax.numpy as jnp
from jax.experimental import pallas as pl


def kernel(write_weights, prev_link, precedence_weights):
    raise NotImplementedError("write your pallas kernel here")


# speedup vs baseline: 2.2664x; 2.2664x over previous
import jax
import jax.numpy as jnp
from jax import lax
from jax.experimental import pallas as pl


def _linkage_body(w_ref, p_ref, prev_ref, link_ref, prec_ref):
    w = w_ref[0, 0]
    p = p_ref[0, 0]
    prev = prev_ref[0, 0]

    m = prev.shape[0]
    wi = w[:, None]
    wj = w[None, :]
    link = (1.0 - wi - wj) * prev + wi * p[None, :]

    ii = lax.broadcasted_iota(jnp.int32, (m, m), 0)
    jj = lax.broadcasted_iota(jnp.int32, (m, m), 1)
    link = jnp.where(ii == jj, 0.0, link)
    link_ref[0, 0] = link

    prec_ref[0, 0] = (1.0 - jnp.sum(w)) * p + w


def kernel(write_weights, prev_link, precedence_weights):
    b, nw, m = write_weights.shape

    grid = (b,)
    vec_spec = pl.BlockSpec((1, nw, m), lambda i: (i, 0, 0))
    mat_spec = pl.BlockSpec((1, nw, m, m), lambda i: (i, 0, 0, 0))

    link, new_prec = pl.pallas_call(
        _linkage_body,
        grid=grid,
        in_specs=[vec_spec, vec_spec, mat_spec],
        out_specs=[mat_spec, vec_spec],
        out_shape=[
            jax.ShapeDtypeStruct(prev_link.shape, prev_link.dtype),
            jax.ShapeDtypeStruct(precedence_weights.shape, precedence_weights.dtype),
        ],
    )(write_weights, precedence_weights, prev_link)
    return (link, new_prec)
